# Initial kernel scaffold; baseline (speedup 1.0000x reference)
import jax
import jax.numpy as jnp
from jax.experimental import pallas as pl


def kernel(inputs, adj, W1, W2):
    raise NotImplementedError("write your pallas kernel here")


# speedup vs baseline: 81.7732x; 81.7732x over previous
import jax
import jax.numpy as jnp
from jax.experimental import pallas as pl
from jax.experimental.pallas import tpu as pltpu

_N = 64
_DIN = 128
_DH = 256
_PB = 32


def _pair_kernel(z_ref, adj_ref, W1a_ref, W1b_ref, W2_ref, out_ref, H0_ref, S_ref):
    step = pl.program_id(0)

    @pl.when(step == 0)
    def _prologue():
        a = adj_ref[...]
        S_ref[...] = jnp.maximum(a, a.T)
        H0_ref[...] = jnp.dot(z_ref[...], W1a_ref[...],
                              preferred_element_type=jnp.float32)

    S = S_ref[...]
    H0 = H0_ref[...]
    w_r = W1b_ref[0, :]
    w_c = W1b_ref[1, :]
    W2v = W2_ref[...].reshape(1, 1, _DH)

    p = step * _PB + jax.lax.broadcasted_iota(jnp.int32, (_PB, 1, 1), 0)
    i = p // _N
    j = p - i * _N
    u = jnp.maximum(i, j)
    l = jnp.minimum(i, j)
    row = jax.lax.broadcasted_iota(jnp.int32, (_PB, _N, _N), 1)
    col = jax.lax.broadcasted_iota(jnp.int32, (_PB, _N, _N), 2)
    msk = ((row < u) & (col < u)) | ((row == u) & (col < l)) | ((row < l) & (col == u))
    P = jnp.maximum(S[None, :, :] * msk.astype(jnp.float32),
                    (row == col).astype(jnp.float32))
    deg = jnp.sum(P, axis=2)
    Dn = jax.lax.rsqrt(jnp.maximum(deg, 1.0))
    C = P * Dn[:, None, :]

    M = jnp.dot(C.reshape(_PB * _N, _N), H0,
                preferred_element_type=jnp.float32).reshape(_PB, _N, _DH)
    oh_i = (col[:, :1, :] == i).astype(jnp.float32)
    oh_j = (col[:, :1, :] == j).astype(jnp.float32)
    ci = jnp.sum(C * oh_i, axis=2)
    cj = jnp.sum(C * oh_j, axis=2)
    M = M + ci[:, :, None] * w_r[None, None, :] + cj[:, :, None] * w_c[None, None, :]
    R = jnp.maximum(M * Dn[:, :, None], 0.0)
    v = jnp.sum(R * W2v, axis=2)
    w = jnp.sum(C * v[:, None, :], axis=2)
    mfac = (oh_i[:, 0, :] + oh_j[:, 0, :]) * Dn
    out_ref[0, 0, :] = jnp.sum(mfac * w, axis=1)


def kernel(inputs, adj, W1, W2):
    W1a = W1[:_DIN]
    W1b = jnp.pad(W1[_DIN:], ((0, 6), (0, 0)))
    W2r = W2.reshape(1, _DH)
    grid = (_N * _N) // _PB
    out = pl.pallas_call(
        _pair_kernel,
        grid=(grid,),
        in_specs=[
            pl.BlockSpec((_N, _DIN), lambda s: (0, 0)),
            pl.BlockSpec((_N, _N), lambda s: (0, 0)),
            pl.BlockSpec((_DIN, _DH), lambda s: (0, 0)),
            pl.BlockSpec((8, _DH), lambda s: (0, 0)),
            pl.BlockSpec((1, _DH), lambda s: (0, 0)),
        ],
        out_specs=pl.BlockSpec((1, 1, _PB), lambda s: (s, 0, 0)),
        out_shape=jax.ShapeDtypeStruct((grid, 1, _PB), jnp.float32),
        scratch_shapes=[pltpu.VMEM((_N, _DH), jnp.float32),
                        pltpu.VMEM((_N, _N), jnp.float32)],
        compiler_params=pltpu.CompilerParams(dimension_semantics=("arbitrary",)),
    )(inputs, adj, W1a, W1b, W2r)
    return out.reshape(-1)
